# Initial kernel scaffold; baseline (speedup 1.0000x reference)
import functools

import jax
import jax.numpy as jnp
from jax import lax
from jax.experimental import pallas as pl
from jax.experimental.pallas import tpu as pltpu
from jax.experimental.pallas import tpu_sc as plsc

_N = 10000
_D = 128
_E = 320000
_NC = 2
_NS = 16
_NW = _NC * _NS
_EB = 128
_BPT = -(-_E // (_NW * _EB))
_EPT = _BPT * _EB
_EPAD = _EPT * _NW
_NPAD = 10240
_RPT = _NPAD // _NS


def _lrelu(a):
    return jnp.where(a > 0, a, 0.2 * a)


def _tc_pre_body(x_ref, w_ref, asrc_ref, adst_ref, h_ref, es_ref, ed_ref, c_ref):
    h = jnp.dot(x_ref[...], w_ref[...], preferred_element_type=jnp.float32)
    h_ref[...] = h
    es = jnp.sum(h * asrc_ref[...], axis=1, keepdims=True)
    ed = jnp.sum(h * adst_ref[...], axis=1, keepdims=True)
    es_ref[...] = es
    ed_ref[...] = ed
    cm = jnp.max(es) + jnp.max(ed)
    c_ref[...] = jnp.full((1, 1), _lrelu(cm), jnp.float32)


def _tc_pre(x, W, asrc, adst):
    return pl.pallas_call(
        _tc_pre_body,
        out_shape=[
            jax.ShapeDtypeStruct((_N, _D), jnp.float32),
            jax.ShapeDtypeStruct((_N, 1), jnp.float32),
            jax.ShapeDtypeStruct((_N, 1), jnp.float32),
            jax.ShapeDtypeStruct((1, 1), jnp.float32),
        ],
    )(x, W, asrc, adst)


def _combine(acc_ref, den_ref, h_ref, es_ref, ed_ref, c_ref, b_ref):
    a = _lrelu(es_ref[...] + ed_ref[...]) - c_ref[...]
    w_self = jnp.exp(a)
    num = acc_ref[0, 0:_N, :] + acc_ref[1, 0:_N, :] + w_self * h_ref[...]
    den = den_ref[0, 0:_N, :] + den_ref[1, 0:_N, :] + w_self
    return num / (den + 1e-16) + b_ref[...]


def _tc_mid_body(acc_ref, den_ref, h_ref, es_ref, ed_ref, c_ref, b_ref,
                 w2_ref, as2_ref, ad2_ref, h2_ref, es2_ref, ed2_ref, c2_ref):
    o1 = _combine(acc_ref, den_ref, h_ref, es_ref, ed_ref, c_ref, b_ref)
    x2 = jnp.where(o1 > 0, o1, jnp.exp(o1) - 1.0)
    h2 = jnp.dot(x2, w2_ref[...], preferred_element_type=jnp.float32)
    h2_ref[...] = h2
    es2 = jnp.sum(h2 * as2_ref[...], axis=1, keepdims=True)
    ed2 = jnp.sum(h2 * ad2_ref[...], axis=1, keepdims=True)
    es2_ref[...] = es2
    ed2_ref[...] = ed2
    cm = jnp.max(es2) + jnp.max(ed2)
    c2_ref[...] = jnp.full((1, 1), _lrelu(cm), jnp.float32)


def _tc_mid(acc, den, h1, es1, ed1, c1, b1, W2, as2, ad2):
    return pl.pallas_call(
        _tc_mid_body,
        out_shape=[
            jax.ShapeDtypeStruct((_N, _D), jnp.float32),
            jax.ShapeDtypeStruct((_N, 1), jnp.float32),
            jax.ShapeDtypeStruct((_N, 1), jnp.float32),
            jax.ShapeDtypeStruct((1, 1), jnp.float32),
        ],
    )(acc, den, h1, es1, ed1, c1, b1, W2, as2, ad2)


def _tc_post_body(acc_ref, den_ref, h_ref, es_ref, ed_ref, c_ref, b_ref, out_ref):
    out_ref[...] = _combine(acc_ref, den_ref, h_ref, es_ref, ed_ref, c_ref, b_ref)


def _tc_post(acc, den, h2, es2, ed2, c2, b2):
    return pl.pallas_call(
        _tc_post_body,
        out_shape=jax.ShapeDtypeStruct((_N, _D), jnp.float32),
    )(acc, den, h2, es2, ed2, c2, b2)


_sc_mesh = plsc.VectorSubcoreMesh(core_axis_name="c", subcore_axis_name="s")


@functools.partial(
    pl.kernel,
    out_type=(
        jax.ShapeDtypeStruct((_NC, _NPAD, _D), jnp.float32),
        jax.ShapeDtypeStruct((_NC, _NPAD), jnp.float32),
    ),
    mesh=_sc_mesh,
    scratch_types=[
        pltpu.VMEM((_N,), jnp.float32),
        pltpu.VMEM((_N,), jnp.float32),
        pltpu.VMEM((_BPT, _EB), jnp.int32),
        pltpu.VMEM((_BPT, _EB), jnp.int32),
        pltpu.VMEM((_EB,), jnp.float32),
        pltpu.VMEM((_EB, _D), jnp.float32),
        pltpu.VMEM((_RPT,), jnp.float32),
        pltpu.VMEM((16,), jnp.float32),
        pltpu.VMEM_SHARED((_NPAD, _D), jnp.float32),
        pltpu.VMEM_SHARED((_NPAD,), jnp.float32),
    ],
)
def _sc_edge(src_hbm, dst_hbm, h_hbm, es_hbm, ed_hbm, c_hbm, acc_hbm, den_hbm,
             es_v, ed_v, src_v, dst_v, expa_v, rows_v, zrow_v, c_v,
             acc_sh, den_sh):
    c = lax.axis_index("c")
    s = lax.axis_index("s")
    wid = c * _NS + s

    pltpu.sync_copy(es_hbm, es_v)
    pltpu.sync_copy(ed_hbm, ed_v)
    pltpu.sync_copy(c_hbm, c_v)
    pltpu.sync_copy(src_hbm.at[pl.ds(wid * _BPT, _BPT)], src_v)
    pltpu.sync_copy(dst_hbm.at[pl.ds(wid * _BPT, _BPT)], dst_v)
    cval = c_v[...]

    zero16 = jnp.zeros((16,), jnp.float32)

    def _zrows(i, _):
        for d in range(_D // 16):
            rows_v[i, pl.ds(d * 16, 16)] = zero16
        return 0

    lax.fori_loop(0, _EB, _zrows, 0)

    def _zden(i, _):
        zrow_v[pl.ds(i * 16, 16)] = zero16
        return 0

    lax.fori_loop(0, _RPT // 16, _zden, 0)

    for k in range(_RPT // _EB):
        pltpu.sync_copy(rows_v, acc_sh.at[pl.ds(s * _RPT + k * _EB, _EB)])
    pltpu.sync_copy(zrow_v, den_sh.at[pl.ds(s * _RPT, _RPT)])
    plsc.subcore_barrier()

    def _block(b, _):
        pltpu.sync_copy(h_hbm.at[src_v.at[b]], rows_v)
        base_eid = wid * _EPT + b * _EB
        for g in range(_EB // 16):
            sidx = src_v[b, pl.ds(g * 16, 16)]
            didx = dst_v[b, pl.ds(g * 16, 16)]
            a = plsc.load_gather(es_v, [sidx]) + plsc.load_gather(ed_v, [didx])
            ex = jnp.exp(_lrelu(a) - cval)
            eid = base_eid + g * 16 + lax.iota(jnp.int32, 16)
            expa_v[pl.ds(g * 16, 16)] = jnp.where(eid < _E, ex, 0.0)

        def _scale(j, _):
            w = plsc.load_gather(expa_v, [jnp.broadcast_to(j, (16,))])
            for d in range(_D // 16):
                sl = pl.ds(d * 16, 16)
                rows_v[j, sl] = rows_v[j, sl] * w
            return 0

        lax.fori_loop(0, _EB, _scale, 0)
        pltpu.sync_copy(rows_v, acc_sh.at[dst_v.at[b]], add=True)
        pltpu.sync_copy(expa_v, den_sh.at[dst_v.at[b]], add=True)
        return 0

    lax.fori_loop(0, _BPT, _block, 0)
    plsc.subcore_barrier()

    for k in range(_RPT // _EB):
        off = s * _RPT + k * _EB
        pltpu.sync_copy(acc_sh.at[pl.ds(off, _EB)], acc_hbm.at[c, pl.ds(off, _EB)])
    pltpu.sync_copy(den_sh.at[pl.ds(s * _RPT, _RPT)], den_hbm.at[c, pl.ds(s * _RPT, _RPT)])


def kernel(x, edge_index, W1, att_src1, att_dst1, bias1,
           W2, att_src2, att_dst2, bias2):
    src = jnp.pad(edge_index[0].astype(jnp.int32), (0, _EPAD - _E))
    dst = jnp.pad(edge_index[1].astype(jnp.int32), (0, _EPAD - _E))
    src = src.reshape(_NW * _BPT, _EB)
    dst = dst.reshape(_NW * _BPT, _EB)
    b1 = bias1.reshape(1, _D)
    b2 = bias2.reshape(1, _D)

    h1, es1, ed1, c1 = _tc_pre(x, W1, att_src1, att_dst1)
    c16 = jnp.broadcast_to(c1.reshape(1), (16,))
    acc1, den1 = _sc_edge(src, dst, h1, es1.reshape(_N), ed1.reshape(_N), c16)
    h2, es2, ed2, c2 = _tc_mid(acc1, den1.reshape(_NC, _NPAD, 1),
                               h1, es1, ed1, c1, b1, W2, att_src2, att_dst2)
    c16b = jnp.broadcast_to(c2.reshape(1), (16,))
    acc2, den2 = _sc_edge(src, dst, h2, es2.reshape(_N), ed2.reshape(_N), c16b)
    return _tc_post(acc2, den2.reshape(_NC, _NPAD, 1), h2, es2, ed2, c2, b2)


# speedup vs baseline: 15.0658x; 15.0658x over previous
import functools

import jax
import jax.numpy as jnp
from jax import lax
from jax.experimental import pallas as pl
from jax.experimental.pallas import tpu as pltpu
from jax.experimental.pallas import tpu_sc as plsc

_N = 10000
_D = 128
_E = 320000
_NC = 2
_NS = 16
_NW = _NC * _NS
_EB = 128
_BPT = 80
_EPT = _BPT * _EB
_EPAD = _EPT * _NW
_NPAD = 10240
_RPT = _NPAD // _NS
_SB = 8


def _lrelu(a):
    return jnp.where(a > 0, a, 0.2 * a)


def _tc_pre_body(x_ref, w_ref, asrc_ref, adst_ref, h_ref, es_ref, ed_ref, c_ref):
    h = jnp.dot(x_ref[...], w_ref[...], preferred_element_type=jnp.float32)
    h_ref[...] = h
    es = jnp.sum(h * asrc_ref[...], axis=1, keepdims=True)
    ed = jnp.sum(h * adst_ref[...], axis=1, keepdims=True)
    es_ref[...] = es
    ed_ref[...] = ed
    cm = jnp.max(es) + jnp.max(ed)
    c_ref[...] = jnp.full((1, 1), _lrelu(cm), jnp.float32)


def _tc_pre(x, W, asrc, adst):
    return pl.pallas_call(
        _tc_pre_body,
        out_shape=[
            jax.ShapeDtypeStruct((_N, _D), jnp.float32),
            jax.ShapeDtypeStruct((_N, 1), jnp.float32),
            jax.ShapeDtypeStruct((_N, 1), jnp.float32),
            jax.ShapeDtypeStruct((1, 1), jnp.float32),
        ],
    )(x, W, asrc, adst)


def _combine(acc_ref, den_ref, h_ref, es_ref, ed_ref, c_ref, b_ref):
    a = _lrelu(es_ref[...] + ed_ref[...]) - c_ref[...]
    w_self = jnp.exp(a)
    num = acc_ref[0, 0:_N, :] + acc_ref[1, 0:_N, :] + w_self * h_ref[...]
    den = den_ref[0, 0:_N, :] + den_ref[1, 0:_N, :] + w_self
    return num / (den + 1e-16) + b_ref[...]


def _tc_combine_body(elu, acc_ref, den_ref, h_ref, es_ref, ed_ref, c_ref, b_ref,
                     out_ref):
    o = _combine(acc_ref, den_ref, h_ref, es_ref, ed_ref, c_ref, b_ref)
    if elu:
        o = jnp.where(o > 0, o, jnp.exp(o) - 1.0)
    out_ref[...] = o


def _tc_combine(acc, den, h, es, ed, c, b, elu):
    return pl.pallas_call(
        functools.partial(_tc_combine_body, elu),
        out_shape=jax.ShapeDtypeStruct((_N, _D), jnp.float32),
    )(acc, den, h, es, ed, c, b)


_sc_mesh = plsc.VectorSubcoreMesh(core_axis_name="c", subcore_axis_name="s")


@functools.partial(
    pl.kernel,
    out_type=(
        jax.ShapeDtypeStruct((_NC, _NPAD, _D), jnp.float32),
        jax.ShapeDtypeStruct((_NC, _NPAD), jnp.float32),
    ),
    mesh=_sc_mesh,
    compiler_params=pltpu.CompilerParams(needs_layout_passes=False),
    scratch_types=[
        pltpu.VMEM((_N,), jnp.float32),
        pltpu.VMEM((_N,), jnp.float32),
        pltpu.VMEM((_SB, _EB), jnp.int32),
        pltpu.VMEM((_SB, _EB), jnp.int32),
        pltpu.VMEM((_EB,), jnp.float32),
        pltpu.VMEM((_EB, _D), jnp.float32),
        pltpu.VMEM((_RPT,), jnp.float32),
        pltpu.VMEM((16,), jnp.float32),
        pltpu.VMEM_SHARED((_NPAD, _D), jnp.float32),
        pltpu.VMEM_SHARED((_NPAD,), jnp.float32),
    ],
)
def _sc_edge(src_hbm, dst_hbm, h_hbm, es_hbm, ed_hbm, c_hbm, acc_hbm, den_hbm,
             es_v, ed_v, src_v, dst_v, expa_v, rows_v, zrow_v, c_v,
             acc_sh, den_sh):
    c = lax.axis_index("c")
    s = lax.axis_index("s")
    wid = c * _NS + s

    pltpu.sync_copy(es_hbm, es_v)
    pltpu.sync_copy(ed_hbm, ed_v)
    pltpu.sync_copy(c_hbm, c_v)
    cval = c_v[...]

    zero16 = jnp.zeros((16,), jnp.float32)

    def _zrows(i, _):
        for d in range(_D // 16):
            rows_v[i, pl.ds(d * 16, 16)] = zero16
        return 0

    lax.fori_loop(0, _EB, _zrows, 0)

    def _zden(i, _):
        zrow_v[pl.ds(i * 16, 16)] = zero16
        return 0

    lax.fori_loop(0, _RPT // 16, _zden, 0)

    for k in range(_RPT // _EB):
        pltpu.sync_copy(rows_v, acc_sh.at[pl.ds(s * _RPT + k * _EB, _EB)])
    pltpu.sync_copy(zrow_v, den_sh.at[pl.ds(s * _RPT, _RPT)])
    plsc.subcore_barrier()

    def _outer(t, _):
        pltpu.sync_copy(src_hbm.at[pl.ds(wid * _BPT + t * _SB, _SB)], src_v)
        pltpu.sync_copy(dst_hbm.at[pl.ds(wid * _BPT + t * _SB, _SB)], dst_v)

        def _block(b, _):
            pltpu.sync_copy(h_hbm.at[src_v.at[b]], rows_v)
            base_eid = wid * _EPT + (t * _SB + b) * _EB
            for g in range(_EB // 16):
                sidx = src_v[b, pl.ds(g * 16, 16)]
                didx = dst_v[b, pl.ds(g * 16, 16)]
                a = plsc.load_gather(es_v, [sidx]) + plsc.load_gather(ed_v, [didx])
                ex = jnp.exp(_lrelu(a) - cval)
                eid = base_eid + g * 16 + lax.iota(jnp.int32, 16)
                expa_v[pl.ds(g * 16, 16)] = jnp.where(eid < _E, ex, 0.0)

            def _scale(j, _):
                w = plsc.load_gather(expa_v, [jnp.broadcast_to(j, (16,))])
                for d in range(_D // 16):
                    sl = pl.ds(d * 16, 16)
                    rows_v[j, sl] = rows_v[j, sl] * w
                return 0

            lax.fori_loop(0, _EB, _scale, 0)
            pltpu.sync_copy(rows_v, acc_sh.at[dst_v.at[b]], add=True)
            pltpu.sync_copy(expa_v, den_sh.at[dst_v.at[b]], add=True)
            return 0

        lax.fori_loop(0, _SB, _block, 0)
        return 0

    lax.fori_loop(0, _BPT // _SB, _outer, 0)
    plsc.subcore_barrier()

    for k in range(_RPT // _EB):
        off = s * _RPT + k * _EB
        pltpu.sync_copy(acc_sh.at[pl.ds(off, _EB)], acc_hbm.at[c, pl.ds(off, _EB)])
    pltpu.sync_copy(den_sh.at[pl.ds(s * _RPT, _RPT)], den_hbm.at[c, pl.ds(s * _RPT, _RPT)])


def kernel(x, edge_index, W1, att_src1, att_dst1, bias1,
           W2, att_src2, att_dst2, bias2):
    src = jnp.pad(edge_index[0].astype(jnp.int32), (0, _EPAD - _E))
    dst = jnp.pad(edge_index[1].astype(jnp.int32), (0, _EPAD - _E))
    src = src.reshape(_NW * _BPT, _EB)
    dst = dst.reshape(_NW * _BPT, _EB)
    b1 = bias1.reshape(1, _D)
    b2 = bias2.reshape(1, _D)

    h1, es1, ed1, c1 = _tc_pre(x, W1, att_src1, att_dst1)
    c16 = jnp.broadcast_to(c1.reshape(1), (16,))
    acc1, den1 = _sc_edge(src, dst, h1, es1.reshape(_N), ed1.reshape(_N), c16)
    x2 = _tc_combine(acc1, den1.reshape(_NC, _NPAD, 1),
                     h1, es1, ed1, c1, b1, elu=True)
    h2, es2, ed2, c2 = _tc_pre(x2, W2, att_src2, att_dst2)
    c16b = jnp.broadcast_to(c2.reshape(1), (16,))
    acc2, den2 = _sc_edge(src, dst, h2, es2.reshape(_N), ed2.reshape(_N), c16b)
    return _tc_combine(acc2, den2.reshape(_NC, _NPAD, 1),
                       h2, es2, ed2, c2, b2, elu=False)


# speedup vs baseline: 17.5131x; 1.1624x over previous
import functools

import jax
import jax.numpy as jnp
from jax import lax
from jax.experimental import pallas as pl
from jax.experimental.pallas import tpu as pltpu
from jax.experimental.pallas import tpu_sc as plsc

_N = 10000
_D = 128
_E = 320000
_NC = 2
_NS = 16
_NW = _NC * _NS
_EB = 64
_BPT = 160
_EPT = _BPT * _EB
_EPAD = _EPT * _NW
_NPAD = 10240
_RPT = _NPAD // _NS
_SB = 32
_NST = _BPT // _SB


def _lrelu(a):
    return jnp.where(a > 0, a, 0.2 * a)


def _tc_pre_body(x_ref, w_ref, asrc_ref, adst_ref, h_ref, es_ref, ed_ref, c_ref):
    h = jnp.dot(x_ref[...], w_ref[...], preferred_element_type=jnp.float32)
    h_ref[...] = h
    es = jnp.sum(h * asrc_ref[...], axis=1, keepdims=True)
    ed = jnp.sum(h * adst_ref[...], axis=1, keepdims=True)
    es_ref[...] = es
    ed_ref[...] = ed
    cm = jnp.max(es) + jnp.max(ed)
    c_ref[...] = jnp.full((1, 1), _lrelu(cm), jnp.float32)


def _tc_pre(x, W, asrc, adst):
    return pl.pallas_call(
        _tc_pre_body,
        out_shape=[
            jax.ShapeDtypeStruct((_N, _D), jnp.float32),
            jax.ShapeDtypeStruct((_N, 1), jnp.float32),
            jax.ShapeDtypeStruct((_N, 1), jnp.float32),
            jax.ShapeDtypeStruct((1, 1), jnp.float32),
        ],
    )(x, W, asrc, adst)


def _combine(acc_ref, den_ref, h_ref, es_ref, ed_ref, c_ref, b_ref):
    a = _lrelu(es_ref[...] + ed_ref[...]) - c_ref[...]
    w_self = jnp.exp(a)
    num = acc_ref[0, 0:_N, :] + acc_ref[1, 0:_N, :] + w_self * h_ref[...]
    den = den_ref[0, 0:_N, :] + den_ref[1, 0:_N, :] + w_self
    return num / (den + 1e-16) + b_ref[...]


def _tc_combine_body(elu, acc_ref, den_ref, h_ref, es_ref, ed_ref, c_ref, b_ref,
                     out_ref):
    o = _combine(acc_ref, den_ref, h_ref, es_ref, ed_ref, c_ref, b_ref)
    if elu:
        o = jnp.where(o > 0, o, jnp.exp(o) - 1.0)
    out_ref[...] = o


def _tc_combine(acc, den, h, es, ed, c, b, elu):
    return pl.pallas_call(
        functools.partial(_tc_combine_body, elu),
        out_shape=jax.ShapeDtypeStruct((_N, _D), jnp.float32),
    )(acc, den, h, es, ed, c, b)


_sc_mesh = plsc.VectorSubcoreMesh(core_axis_name="c", subcore_axis_name="s")


@functools.partial(
    pl.kernel,
    out_type=(
        jax.ShapeDtypeStruct((_NC, _NPAD, _D), jnp.float32),
        jax.ShapeDtypeStruct((_NC, _NPAD), jnp.float32),
    ),
    mesh=_sc_mesh,
    compiler_params=pltpu.CompilerParams(needs_layout_passes=False),
    scratch_types=[
        pltpu.VMEM((_N,), jnp.float32),
        pltpu.VMEM((_N,), jnp.float32),
        pltpu.VMEM((_SB, _EB), jnp.int32),
        pltpu.VMEM((_SB, _EB), jnp.int32),
        pltpu.VMEM((_EB,), jnp.float32),
        pltpu.VMEM((_EB,), jnp.float32),
        pltpu.VMEM((_EB, _D), jnp.float32),
        pltpu.VMEM((_EB, _D), jnp.float32),
        pltpu.VMEM((_RPT,), jnp.float32),
        pltpu.VMEM((16,), jnp.float32),
        pltpu.VMEM_SHARED((_NPAD, _D), jnp.float32),
        pltpu.VMEM_SHARED((_NPAD,), jnp.float32),
        pltpu.SemaphoreType.DMA,
        pltpu.SemaphoreType.DMA,
        pltpu.SemaphoreType.DMA,
        pltpu.SemaphoreType.DMA,
        pltpu.SemaphoreType.DMA,
        pltpu.SemaphoreType.DMA,
    ],
)
def _sc_edge(src_hbm, dst_hbm, h_hbm, es_hbm, ed_hbm, c_hbm, acc_hbm, den_hbm,
             es_v, ed_v, src_v, dst_v, expa0, expa1, rows0, rows1, zrow_v, c_v,
             acc_sh, den_sh, sg0, sg1, sr0, sr1, sd0, sd1):
    c = lax.axis_index("c")
    s = lax.axis_index("s")
    wid = c * _NS + s
    rows = (rows0, rows1)
    expa = (expa0, expa1)
    sg = (sg0, sg1)
    sr = (sr0, sr1)
    sd = (sd0, sd1)

    pltpu.sync_copy(es_hbm, es_v)
    pltpu.sync_copy(ed_hbm, ed_v)
    pltpu.sync_copy(c_hbm, c_v)
    cval = c_v[...]

    zero16 = jnp.zeros((16,), jnp.float32)

    def _zrows(i, _):
        for d in range(_D // 16):
            rows0[i, pl.ds(d * 16, 16)] = zero16
        return 0

    lax.fori_loop(0, _EB, _zrows, 0)

    def _zden(i, _):
        zrow_v[pl.ds(i * 16, 16)] = zero16
        return 0

    lax.fori_loop(0, _RPT // 16, _zden, 0)

    for k in range(_RPT // _EB):
        pltpu.sync_copy(rows0, acc_sh.at[pl.ds(s * _RPT + k * _EB, _EB)])
    pltpu.sync_copy(zrow_v, den_sh.at[pl.ds(s * _RPT, _RPT)])
    plsc.subcore_barrier()

    def _expa_block(b, t, eb):
        base_eid = wid * _EPT + (t * _SB + b) * _EB
        for g in range(_EB // 16):
            sidx = src_v[b, pl.ds(g * 16, 16)]
            didx = dst_v[b, pl.ds(g * 16, 16)]
            a = plsc.load_gather(es_v, [sidx]) + plsc.load_gather(ed_v, [didx])
            ex = jnp.exp(_lrelu(a) - cval)
            eid = base_eid + g * 16 + lax.iota(jnp.int32, 16)
            eb[pl.ds(g * 16, 16)] = jnp.where(eid < _E, ex, 0.0)

    def _scale_block(rb, eb):
        def _scale(j, _):
            w = plsc.load_gather(eb, [jnp.broadcast_to(j, (16,))])
            for d in range(_D // 16):
                sl = pl.ds(d * 16, 16)
                rb[j, sl] = rb[j, sl] * w
            return 0

        lax.fori_loop(0, _EB, _scale, 0)

    def _stage(t, _):
        pltpu.sync_copy(src_hbm.at[pl.ds(wid * _BPT + t * _SB, _SB)], src_v)
        pltpu.sync_copy(dst_hbm.at[pl.ds(wid * _BPT + t * _SB, _SB)], dst_v)
        gh = [None, None]
        rh = [None, None]
        dh = [None, None]
        gh[0] = pltpu.async_copy(h_hbm.at[src_v.at[0]], rows0, sg0)
        for b in range(_SB):
            u = b & 1
            v = 1 - u
            if b >= 1:
                rh[v].wait()
                dh[v].wait()
            _expa_block(b, t, expa[u])
            if b + 1 < _SB:
                gh[v] = pltpu.async_copy(
                    h_hbm.at[src_v.at[b + 1]], rows[v], sg[v])
            gh[u].wait()
            _scale_block(rows[u], expa[u])
            rh[u] = pltpu.async_copy(rows[u], acc_sh.at[dst_v.at[b]], sr[u],
                                     add=True)
            dh[u] = pltpu.async_copy(expa[u], den_sh.at[dst_v.at[b]], sd[u],
                                     add=True)
        ul = (_SB - 1) & 1
        rh[ul].wait()
        dh[ul].wait()
        return 0

    lax.fori_loop(0, _NST, _stage, 0)
    plsc.subcore_barrier()

    for k in range(_RPT // _EB):
        off = s * _RPT + k * _EB
        pltpu.sync_copy(acc_sh.at[pl.ds(off, _EB)], acc_hbm.at[c, pl.ds(off, _EB)])
    pltpu.sync_copy(den_sh.at[pl.ds(s * _RPT, _RPT)], den_hbm.at[c, pl.ds(s * _RPT, _RPT)])


def kernel(x, edge_index, W1, att_src1, att_dst1, bias1,
           W2, att_src2, att_dst2, bias2):
    src = jnp.pad(edge_index[0].astype(jnp.int32), (0, _EPAD - _E))
    dst = jnp.pad(edge_index[1].astype(jnp.int32), (0, _EPAD - _E))
    src = src.reshape(_NW * _BPT, _EB)
    dst = dst.reshape(_NW * _BPT, _EB)
    b1 = bias1.reshape(1, _D)
    b2 = bias2.reshape(1, _D)

    h1, es1, ed1, c1 = _tc_pre(x, W1, att_src1, att_dst1)
    c16 = jnp.broadcast_to(c1.reshape(1), (16,))
    acc1, den1 = _sc_edge(src, dst, h1, es1.reshape(_N), ed1.reshape(_N), c16)
    x2 = _tc_combine(acc1, den1.reshape(_NC, _NPAD, 1),
                     h1, es1, ed1, c1, b1, elu=True)
    h2, es2, ed2, c2 = _tc_pre(x2, W2, att_src2, att_dst2)
    c16b = jnp.broadcast_to(c2.reshape(1), (16,))
    acc2, den2 = _sc_edge(src, dst, h2, es2.reshape(_N), ed2.reshape(_N), c16b)
    return _tc_combine(acc2, den2.reshape(_NC, _NPAD, 1),
                       h2, es2, ed2, c2, b2, elu=False)
